# initial kernel scaffold (unmeasured)
import jax
import jax.numpy as jnp
from jax import lax
from jax.experimental import pallas as pl
from jax.experimental.pallas import tpu as pltpu

N_DEV = 32


def kernel(x, w_mat):
    m_total, k_local = x.shape
    k_total, n_out = w_mat.shape
    m_blk = m_total // N_DEV
    assert m_blk == k_local and k_total == m_total

    def body(x_ref, w_ref, out_ref, xb_ref, buf_ref, wbuf_ref,
             send_sems, recv_sems, wcopy_sems):
        my = lax.axis_index("i")

        def w_slot(s):
            return (my - s) % N_DEV

        def w_copy(g, slot):
            s = w_slot(g)
            return pltpu.make_async_copy(
                w_ref.at[pl.ds(s * m_blk, m_blk), :],
                wbuf_ref.at[slot],
                wcopy_sems.at[slot],
            )

        w_copy(0, 0).start()

        xb_ref[...] = x_ref[...].astype(jnp.bfloat16)

        barrier_sem = pltpu.get_barrier_semaphore()
        for d in range(1, N_DEV):
            pl.semaphore_signal(
                barrier_sem, inc=1,
                device_id=((my + d) % N_DEV,),
                device_id_type=pl.DeviceIdType.MESH,
            )
        pl.semaphore_wait(barrier_sem, N_DEV - 1)

        sends = []
        for d in range(1, N_DEV):
            tgt = (my + d) % N_DEV
            rdma = pltpu.make_async_remote_copy(
                src_ref=xb_ref.at[pl.ds(tgt * m_blk, m_blk), :],
                dst_ref=buf_ref.at[pl.ds(my * m_blk, m_blk), :],
                send_sem=send_sems.at[d - 1],
                recv_sem=recv_sems.at[d - 1],
                device_id=(tgt,),
                device_id_type=pl.DeviceIdType.MESH,
            )
            rdma.start()
            sends.append(rdma)

        acc = None
        for g in range(N_DEV):
            slot = g % 2
            if g + 1 < N_DEV:
                w_copy(g + 1, 1 - slot).start()
            w_copy(g, slot).wait()
            s = w_slot(g)
            if g == 0:
                a = xb_ref[pl.ds(my * m_blk, m_blk), :]
            else:
                recv = pltpu.make_async_remote_copy(
                    src_ref=xb_ref.at[pl.ds(0, m_blk), :],
                    dst_ref=buf_ref.at[pl.ds(s * m_blk, m_blk), :],
                    send_sem=send_sems.at[g - 1],
                    recv_sem=recv_sems.at[g - 1],
                    device_id=(my,),
                    device_id_type=pl.DeviceIdType.MESH,
                )
                recv.wait_recv()
                a = buf_ref[pl.ds(s * m_blk, m_blk), :]
            wv = wbuf_ref[slot].astype(jnp.bfloat16)
            part = jnp.dot(a, wv, preferred_element_type=jnp.float32)
            acc = part if acc is None else acc + part
        out_ref[...] = acc

        for rdma in sends:
            rdma.wait_send()

    return pl.pallas_call(
        body,
        out_shape=jax.ShapeDtypeStruct((m_blk, n_out), jnp.float32),
        in_specs=[
            pl.BlockSpec(memory_space=pltpu.VMEM),
            pl.BlockSpec(memory_space=pltpu.ANY),
        ],
        out_specs=pl.BlockSpec(memory_space=pltpu.VMEM),
        scratch_shapes=[
            pltpu.VMEM((m_total, k_local), jnp.bfloat16),
            pltpu.VMEM((m_total, k_local), jnp.bfloat16),
            pltpu.VMEM((2, m_blk, n_out), jnp.float32),
            pltpu.SemaphoreType.DMA((N_DEV - 1,)),
            pltpu.SemaphoreType.DMA((N_DEV - 1,)),
            pltpu.SemaphoreType.DMA((2,)),
        ],
        compiler_params=pltpu.CompilerParams(collective_id=0),
    )(x, w_mat)


# baseline (device time: 85120 ns/iter reference)
import jax
import jax.numpy as jnp
from jax import lax
from jax.experimental import pallas as pl
from jax.experimental.pallas import tpu as pltpu

N_DEV = 32


def kernel(x, w_mat):
    m_total, k_local = x.shape
    k_total, n_out = w_mat.shape
    m_blk = m_total // N_DEV
    assert m_blk == k_local and k_total == m_total

    def body(x_ref, w_ref, out_ref, xb_ref, buf_ref, wbuf_ref,
             send_sems, recv_sems, wcopy_sems):
        my = lax.axis_index("i")

        def w_slot(s):
            return (my - s) % N_DEV

        def w_copy(g, slot):
            s = w_slot(g)
            return pltpu.make_async_copy(
                w_ref.at[pl.ds(s * m_blk, m_blk), :],
                wbuf_ref.at[slot],
                wcopy_sems.at[slot],
            )

        w_copy(0, 0).start()

        xb_ref[...] = x_ref[...].astype(jnp.bfloat16)

        barrier_sem = pltpu.get_barrier_semaphore()
        for d in range(1, N_DEV):
            pl.semaphore_signal(
                barrier_sem, inc=1,
                device_id=((my + d) % N_DEV,),
                device_id_type=pl.DeviceIdType.MESH,
            )
        pl.semaphore_wait(barrier_sem, N_DEV - 1)

        sends = []
        for d in range(1, N_DEV):
            tgt = (my + d) % N_DEV
            rdma = pltpu.make_async_remote_copy(
                src_ref=xb_ref.at[pl.ds(tgt * m_blk, m_blk), :],
                dst_ref=buf_ref.at[pl.ds(my * m_blk, m_blk), :],
                send_sem=send_sems.at[d - 1],
                recv_sem=recv_sems.at[d - 1],
                device_id=(tgt,),
                device_id_type=pl.DeviceIdType.MESH,
            )
            rdma.start()
            sends.append(rdma)

        for g in range(N_DEV):
            slot = g % 2
            if g + 1 < N_DEV:
                w_copy(g + 1, 1 - slot).start()
            w_copy(g, slot).wait()
            s = w_slot(g)
            if g == 0:
                a = xb_ref[pl.ds(my * m_blk, m_blk), :]
            else:
                recv = pltpu.make_async_remote_copy(
                    src_ref=xb_ref.at[pl.ds(0, m_blk), :],
                    dst_ref=buf_ref.at[pl.ds(s * m_blk, m_blk), :],
                    send_sem=send_sems.at[g - 1],
                    recv_sem=recv_sems.at[g - 1],
                    device_id=(my,),
                    device_id_type=pl.DeviceIdType.MESH,
                )
                recv.wait_recv()
                a = buf_ref[pl.ds(s * m_blk, m_blk), :]
            wv = wbuf_ref[slot].astype(jnp.bfloat16)
            part = jnp.dot(a, wv, preferred_element_type=jnp.float32)
            if g == 0:
                out_ref[...] = part
            else:
                out_ref[...] += part

        for rdma in sends:
            rdma.wait_send()

    return pl.pallas_call(
        body,
        out_shape=jax.ShapeDtypeStruct((m_blk, n_out), jnp.float32),
        in_specs=[
            pl.BlockSpec(memory_space=pltpu.VMEM),
            pl.BlockSpec(memory_space=pltpu.MemorySpace.HBM),
        ],
        out_specs=pl.BlockSpec(memory_space=pltpu.VMEM),
        scratch_shapes=[
            pltpu.VMEM((m_total, k_local), jnp.bfloat16),
            pltpu.VMEM((m_total, k_local), jnp.bfloat16),
            pltpu.VMEM((2, m_blk, n_out), jnp.float32),
            pltpu.SemaphoreType.DMA((N_DEV - 1,)),
            pltpu.SemaphoreType.DMA((N_DEV - 1,)),
            pltpu.SemaphoreType.DMA((2,)),
        ],
        compiler_params=pltpu.CompilerParams(collective_id=0),
    )(x, w_mat)
